# baseline (speedup 1.0000x reference)
import functools
import math

import jax
import jax.numpy as jnp
from jax import lax
from jax.experimental import pallas as pl
from jax.experimental.pallas import tpu as pltpu
from jax.experimental.pallas import tpu_sc as plsc

D = 128
SCALE = math.sqrt(D)

NC = 2
NS = 16
NW = NC * NS

CH = 128
NBUF = 5
LAG = 3
L = 16
RU = 4


@functools.partial(jax.jit, static_argnames=("b_total",))
def _sc_gather(x_grp, table, b_total):
    g_per_w = x_grp.shape[1]
    b_per_w = g_per_w * CH
    niter = g_per_w // NBUF

    mesh = plsc.VectorSubcoreMesh(core_axis_name="c", subcore_axis_name="s")

    @functools.partial(
        pl.kernel,
        mesh=mesh,
        out_type=jax.ShapeDtypeStruct((b_total, D), jnp.float32),
        scratch_types=[
            pltpu.VMEM((g_per_w, CH), jnp.int32),
            pltpu.VMEM((NBUF, CH, D), jnp.float32),
        ]
        + [pltpu.SemaphoreType.DMA] * (2 * NBUF),
    )
    def k(x_hbm, tab_hbm, out_hbm, idx_v, bufs, *sems):
        gsems, osems = sems[:NBUF], sems[NBUF:]
        wid = lax.axis_index("s") * NC + lax.axis_index("c")
        base = wid * b_per_w
        pltpu.sync_copy(x_hbm.at[wid], idx_v)

        def g_issue(g, b):
            pltpu.async_copy(tab_hbm.at[idx_v.at[g]], bufs.at[b], gsems[b])

        def g_wait(g, b):
            pltpu.make_async_copy(
                tab_hbm.at[idx_v.at[g]], bufs.at[b], gsems[b]
            ).wait()

        def s_issue(g, b):
            pltpu.async_copy(
                bufs.at[b], out_hbm.at[pl.ds(base + g * CH, CH)], osems[b]
            )

        def s_wait(g, b):
            pltpu.make_async_copy(
                bufs.at[b], out_hbm.at[pl.ds(base + g * CH, CH)], osems[b]
            ).wait()

        def scale_buf(b):
            def sbody(r, _):
                for u in range(RU):
                    for c in range(D // L):
                        sl = pl.ds(c * L, L)
                        bufs[b, r * RU + u, sl] = bufs[b, r * RU + u, sl] * SCALE
                return 0

            lax.fori_loop(0, CH // RU, sbody, 0, unroll=4)

        for b in range(NBUF):
            g_issue(b, b)
        for b in range(NBUF - LAG):
            g_wait(b, b)
            scale_buf(b)
            s_issue(b, b)

        def outer(i, _):
            for b in range(NBUF):
                t = i * NBUF + b
                s_wait(t - NBUF, b)
                g_issue(t, b)
                bl = (b - LAG) % NBUF
                g_wait(t - LAG, bl)
                scale_buf(bl)
                s_issue(t - LAG, bl)
            return 0

        lax.fori_loop(1, niter, outer, 0)

        gl = g_per_w
        for t in range(gl - LAG, gl):
            b = t % NBUF
            g_wait(t, b)
            scale_buf(b)
            s_issue(t, b)
        for t in range(gl - NBUF, gl):
            s_wait(t, t % NBUF)

    return k(x_grp, table)


def kernel(x, table):
    bs, sl = x.shape
    b_total = bs * sl
    x_grp = x.reshape(NW, b_total // (NW * CH), CH).astype(jnp.int32)
    out = _sc_gather(x_grp, table, b_total)
    return out.reshape(bs, sl, D)


# speedup vs baseline: 1.0099x; 1.0099x over previous
import functools
import math

import jax
import jax.numpy as jnp
from jax import lax
from jax.experimental import pallas as pl
from jax.experimental.pallas import tpu as pltpu
from jax.experimental.pallas import tpu_sc as plsc

D = 128
SCALE = math.sqrt(D)

NC = 2
NS = 16
NW = NC * NS

CH = 128
NBUF = 5
LAG = 3
L = 16
RU = 2


@functools.partial(jax.jit, static_argnames=("b_total",))
def _sc_gather(x_grp, table, b_total):
    g_per_w = x_grp.shape[1]
    b_per_w = g_per_w * CH
    niter = g_per_w // NBUF

    mesh = plsc.VectorSubcoreMesh(core_axis_name="c", subcore_axis_name="s")

    @functools.partial(
        pl.kernel,
        mesh=mesh,
        out_type=jax.ShapeDtypeStruct((b_total, D), jnp.float32),
        scratch_types=[
            pltpu.VMEM((g_per_w, CH), jnp.int32),
            pltpu.VMEM((NBUF, CH, D), jnp.float32),
        ]
        + [pltpu.SemaphoreType.DMA] * (2 * NBUF),
    )
    def k(x_hbm, tab_hbm, out_hbm, idx_v, bufs, *sems):
        gsems, osems = sems[:NBUF], sems[NBUF:]
        wid = lax.axis_index("s") * NC + lax.axis_index("c")
        base = wid * b_per_w
        pltpu.sync_copy(x_hbm.at[wid], idx_v)

        def g_issue(g, b):
            pltpu.async_copy(tab_hbm.at[idx_v.at[g]], bufs.at[b], gsems[b])

        def g_wait(g, b):
            pltpu.make_async_copy(
                tab_hbm.at[idx_v.at[g]], bufs.at[b], gsems[b]
            ).wait()

        def s_issue(g, b):
            pltpu.async_copy(
                bufs.at[b], out_hbm.at[pl.ds(base + g * CH, CH)], osems[b]
            )

        def s_wait(g, b):
            pltpu.make_async_copy(
                bufs.at[b], out_hbm.at[pl.ds(base + g * CH, CH)], osems[b]
            ).wait()

        def scale_buf(b):
            def sbody(r, _):
                for u in range(RU):
                    for c in range(D // L):
                        sl = pl.ds(c * L, L)
                        bufs[b, r * RU + u, sl] = bufs[b, r * RU + u, sl] * SCALE
                return 0

            lax.fori_loop(0, CH // RU, sbody, 0, unroll=2)

        for b in range(NBUF):
            g_issue(b, b)
        for b in range(NBUF - LAG):
            g_wait(b, b)
            scale_buf(b)
            s_issue(b, b)

        def outer(i, _):
            for b in range(NBUF):
                t = i * NBUF + b
                s_wait(t - NBUF, b)
                g_issue(t, b)
                bl = (b - LAG) % NBUF
                g_wait(t - LAG, bl)
                scale_buf(bl)
                s_issue(t - LAG, bl)
            return 0

        lax.fori_loop(1, niter, outer, 0)

        gl = g_per_w
        for t in range(gl - LAG, gl):
            b = t % NBUF
            g_wait(t, b)
            scale_buf(b)
            s_issue(t, b)
        for t in range(gl - NBUF, gl):
            s_wait(t, t % NBUF)

    return k(x_grp, table)


def kernel(x, table):
    bs, sl = x.shape
    b_total = bs * sl
    x_grp = x.reshape(NW, b_total // (NW * CH), CH).astype(jnp.int32)
    out = _sc_gather(x_grp, table, b_total)
    return out.reshape(bs, sl, D)
